# Initial kernel scaffold; baseline (speedup 1.0000x reference)
import jax
import jax.numpy as jnp
from jax.experimental import pallas as pl


def kernel(x, edge_index, batch, W_in, b_in, W_h0, b_h0, W_out, b_out):
    raise NotImplementedError("write your pallas kernel here")


# speedup vs baseline: 18.8626x; 18.8626x over previous
import functools

import jax
import jax.numpy as jnp
from jax import lax
from jax.experimental import pallas as pl
from jax.experimental.pallas import tpu as pltpu
from jax.experimental.pallas import tpu_sc as plsc

NN = 10000
NE = 320000
DH = 128
DO = 64
NG = 64

NC = 2
NS = 16
NW = NC * NS
EPT = NE // NW
CH = 40
NCH = EPT // CH
SCH = 50
SEGC = 8
NSEG = EPT // (SEGC * SCH)
SLAB = 624
TAILO = SLAB * NS
TAILN = NN - TAILO
DEGW = 128

_MESH = plsc.VectorSubcoreMesh(
    core_axis_name="c", subcore_axis_name="s", num_cores=NC, num_subcores=NS)


@functools.partial(
    pl.kernel,
    out_type=jax.ShapeDtypeStruct((NC, NN, DEGW), jnp.float32),
    mesh=_MESH,
    scratch_types=[
        pltpu.VMEM((NCH, CH), jnp.int32),
        pltpu.VMEM((CH, DEGW), jnp.float32),
        pltpu.VMEM_SHARED((NN, DEGW), jnp.float32),
    ],
)
def _deg_kernel(dst_hbm, ones_hbm, zer_hbm, out_hbm, idx_v, ones_v, deg_sh):
    cid = lax.axis_index("c")
    sid = lax.axis_index("s")
    wid = cid * NS + sid
    pltpu.sync_copy(zer_hbm.at[pl.ds(0, SLAB)],
                    deg_sh.at[pl.ds(sid * SLAB, SLAB)])

    @pl.when(sid == NS - 1)
    def _init_tail():
        pltpu.sync_copy(zer_hbm.at[pl.ds(0, TAILN)],
                        deg_sh.at[pl.ds(TAILO, TAILN)])

    pltpu.sync_copy(dst_hbm.at[wid], idx_v)
    pltpu.sync_copy(ones_hbm, ones_v)
    plsc.subcore_barrier()

    def body(j, carry):
        pltpu.sync_copy(ones_v, deg_sh.at[idx_v.at[j]], add=True)
        return carry

    lax.fori_loop(0, NCH, body, 0)
    plsc.subcore_barrier()
    pltpu.sync_copy(deg_sh.at[pl.ds(sid * SLAB, SLAB)],
                    out_hbm.at[cid, pl.ds(sid * SLAB, SLAB)])

    @pl.when(sid == NS - 1)
    def _out_tail():
        pltpu.sync_copy(deg_sh.at[pl.ds(TAILO, TAILN)],
                        out_hbm.at[cid, pl.ds(TAILO, TAILN)])


@functools.partial(
    pl.kernel,
    out_type=jax.ShapeDtypeStruct((NC, NN, DH), jnp.float32),
    mesh=_MESH,
    scratch_types=[
        pltpu.VMEM((SEGC, SCH), jnp.int32),
        pltpu.VMEM((SEGC, SCH), jnp.int32),
        pltpu.VMEM((SCH, DH), jnp.float32),
        pltpu.VMEM((SCH, DH), jnp.float32),
        pltpu.SemaphoreType.DMA,
        pltpu.SemaphoreType.DMA,
        pltpu.VMEM_SHARED((NN, DH), jnp.float32),
    ],
)
def _scatter_kernel(g_hbm, src_hbm, dst_hbm, zer_hbm, out_hbm,
                    srcseg, dstseg, rows0, rows1, sem0, sem1, s_sh):
    cid = lax.axis_index("c")
    sid = lax.axis_index("s")
    wid = cid * NS + sid
    pltpu.sync_copy(zer_hbm.at[pl.ds(0, SLAB)],
                    s_sh.at[pl.ds(sid * SLAB, SLAB)])

    @pl.when(sid == NS - 1)
    def _init_tail():
        pltpu.sync_copy(zer_hbm.at[pl.ds(0, TAILN)],
                        s_sh.at[pl.ds(TAILO, TAILN)])

    plsc.subcore_barrier()

    def seg_body(t, carry):
        pltpu.sync_copy(src_hbm.at[wid, t], srcseg)
        pltpu.sync_copy(dst_hbm.at[wid, t], dstseg)
        pltpu.async_copy(g_hbm.at[srcseg.at[0]], rows0, sem0)
        for j in range(SEGC):
            rows, sem = (rows0, sem0) if j % 2 == 0 else (rows1, sem1)
            if j + 1 < SEGC:
                nrows, nsem = (rows1, sem1) if j % 2 == 0 else (rows0, sem0)
                pltpu.async_copy(g_hbm.at[srcseg.at[j + 1]], nrows, nsem)
            pltpu.make_async_copy(g_hbm.at[srcseg.at[j]], rows, sem).wait()
            pltpu.sync_copy(rows, s_sh.at[dstseg.at[j]], add=True)
        return carry

    lax.fori_loop(0, NSEG, seg_body, 0)
    plsc.subcore_barrier()
    pltpu.sync_copy(s_sh.at[pl.ds(sid * SLAB, SLAB)],
                    out_hbm.at[cid, pl.ds(sid * SLAB, SLAB)])

    @pl.when(sid == NS - 1)
    def _out_tail():
        pltpu.sync_copy(s_sh.at[pl.ds(TAILO, TAILN)],
                        out_hbm.at[cid, pl.ds(TAILO, TAILN)])


_RB = 2000
_NB = NN // _RB


def _mm_scale_body(dega_ref, degb_ref, x_ref, w_ref, g_ref, dinv_ref):
    deg = dega_ref[:, 0:1] + degb_ref[:, 0:1] + 1.0
    di = lax.rsqrt(deg)
    g_ref[...] = jnp.dot(x_ref[...], w_ref[...],
                         preferred_element_type=jnp.float32) * di
    dinv_ref[...] = di


def _mm_scale(dega, degb, x, w):
    return pl.pallas_call(
        _mm_scale_body,
        grid=(_NB,),
        in_specs=[
            pl.BlockSpec((_RB, DEGW), lambda i: (i, 0)),
            pl.BlockSpec((_RB, DEGW), lambda i: (i, 0)),
            pl.BlockSpec((_RB, DH), lambda i: (i, 0)),
            pl.BlockSpec((DH, DH), lambda i: (0, 0)),
        ],
        out_specs=[
            pl.BlockSpec((_RB, DH), lambda i: (i, 0)),
            pl.BlockSpec((_RB, 1), lambda i: (i, 0)),
        ],
        out_shape=[
            jax.ShapeDtypeStruct((NN, DH), jnp.float32),
            jax.ShapeDtypeStruct((NN, 1), jnp.float32),
        ],
    )(dega, degb, x, w)


def _mid_body(s0_ref, s1_ref, g1_ref, dinv_ref, b_ref, w_ref, g2_ref):
    di = dinv_ref[...]
    h1 = jnp.maximum(
        di * (s0_ref[...] + s1_ref[...] + g1_ref[...]) + b_ref[...], 0.0)
    g2_ref[...] = jnp.dot(h1, w_ref[...],
                          preferred_element_type=jnp.float32) * di


def _mid(s0, s1, g1, dinv, b, w):
    return pl.pallas_call(
        _mid_body,
        grid=(_NB,),
        in_specs=[
            pl.BlockSpec((_RB, DH), lambda i: (i, 0)),
            pl.BlockSpec((_RB, DH), lambda i: (i, 0)),
            pl.BlockSpec((_RB, DH), lambda i: (i, 0)),
            pl.BlockSpec((_RB, 1), lambda i: (i, 0)),
            pl.BlockSpec((1, DH), lambda i: (0, 0)),
            pl.BlockSpec((DH, DH), lambda i: (0, 0)),
        ],
        out_specs=pl.BlockSpec((_RB, DH), lambda i: (i, 0)),
        out_shape=jax.ShapeDtypeStruct((NN, DH), jnp.float32),
    )(s0, s1, g1, dinv, b, w)


def _final_body(s0_ref, s1_ref, g2_ref, dinv_ref, b_ref, batch_ref,
                wout_ref, bout_ref, out_ref, acc, cnt):
    i = pl.program_id(0)

    @pl.when(i == 0)
    def _init():
        acc[...] = jnp.zeros_like(acc)
        cnt[...] = jnp.zeros_like(cnt)

    h2 = jnp.maximum(
        dinv_ref[...] * (s0_ref[...] + s1_ref[...] + g2_ref[...])
        + b_ref[...], 0.0)
    bt = batch_ref[0]
    m = (bt == lax.broadcasted_iota(jnp.int32, (NG, _RB), 0)
         ).astype(jnp.float32)
    acc[...] += jnp.dot(m, h2, preferred_element_type=jnp.float32)
    cnt[...] += jnp.sum(m, axis=1, keepdims=True)

    @pl.when(i == _NB - 1)
    def _fin():
        pooled = acc[...] / jnp.maximum(cnt[...], 1.0)
        out_ref[...] = jnp.dot(pooled, wout_ref[...],
                               preferred_element_type=jnp.float32) + bout_ref[...]


def _final(s0, s1, g2, dinv, b, batch3, wout, bout):
    return pl.pallas_call(
        _final_body,
        grid=(_NB,),
        in_specs=[
            pl.BlockSpec((_RB, DH), lambda i: (i, 0)),
            pl.BlockSpec((_RB, DH), lambda i: (i, 0)),
            pl.BlockSpec((_RB, DH), lambda i: (i, 0)),
            pl.BlockSpec((_RB, 1), lambda i: (i, 0)),
            pl.BlockSpec((1, DH), lambda i: (0, 0)),
            pl.BlockSpec((1, 1, _RB), lambda i: (i, 0, 0)),
            pl.BlockSpec((DH, DO), lambda i: (0, 0)),
            pl.BlockSpec((1, DO), lambda i: (0, 0)),
        ],
        out_specs=pl.BlockSpec((NG, DO), lambda i: (0, 0)),
        out_shape=jax.ShapeDtypeStruct((NG, DO), jnp.float32),
        scratch_shapes=[
            pltpu.VMEM((NG, DH), jnp.float32),
            pltpu.VMEM((NG, 1), jnp.float32),
        ],
    )(s0, s1, g2, dinv, b, batch3, wout, bout)


def kernel(x, edge_index, batch, W_in, b_in, W_h0, b_h0, W_out, b_out):
    src4 = edge_index[0].reshape(NW, NSEG, SEGC, SCH)
    dst4 = edge_index[1].reshape(NW, NSEG, SEGC, SCH)
    dst3 = edge_index[1].reshape(NW, NCH, CH)
    ones_deg = jnp.ones((CH, DEGW), jnp.float32)
    zer_deg = jnp.zeros((SLAB, DEGW), jnp.float32)
    zer_s = jnp.zeros((SLAB, DH), jnp.float32)

    deg2 = _deg_kernel(dst3, ones_deg, zer_deg)
    g1, dinv = _mm_scale(deg2[0], deg2[1], x, W_in)
    s1 = _scatter_kernel(g1, src4, dst4, zer_s)
    g2 = _mid(s1[0], s1[1], g1, dinv, b_in.reshape(1, DH), W_h0)
    s2 = _scatter_kernel(g2, src4, dst4, zer_s)
    return _final(s2[0], s2[1], g2, dinv, b_h0.reshape(1, DH),
                  batch.reshape(_NB, 1, _RB), W_out, b_out.reshape(1, DO))


# speedup vs baseline: 24.3212x; 1.2894x over previous
import functools

import jax
import jax.numpy as jnp
from jax import lax
from jax.experimental import pallas as pl
from jax.experimental.pallas import tpu as pltpu
from jax.experimental.pallas import tpu_sc as plsc

NN = 10000
NE = 320000
DH = 128
DO = 64
NG = 64

NC = 2
NS = 16
NW = NC * NS
EPT = NE // NW
CH = 40
NCH = EPT // CH
SCH = 100
SEGC = 10
NSEG = EPT // (SEGC * SCH)
SLAB = 624
TAILO = SLAB * NS
TAILN = NN - TAILO
DEGW = 128

_MESH = plsc.VectorSubcoreMesh(
    core_axis_name="c", subcore_axis_name="s", num_cores=NC, num_subcores=NS)


@functools.partial(
    pl.kernel,
    out_type=jax.ShapeDtypeStruct((NC, NN, DEGW), jnp.float32),
    mesh=_MESH,
    scratch_types=[
        pltpu.VMEM((NCH, CH), jnp.int32),
        pltpu.VMEM((CH, DEGW), jnp.float32),
        pltpu.VMEM_SHARED((NN, DEGW), jnp.float32),
    ],
)
def _deg_kernel(dst_hbm, ones_hbm, zer_hbm, out_hbm, idx_v, ones_v, deg_sh):
    cid = lax.axis_index("c")
    sid = lax.axis_index("s")
    wid = cid * NS + sid
    pltpu.sync_copy(zer_hbm.at[pl.ds(0, SLAB)],
                    deg_sh.at[pl.ds(sid * SLAB, SLAB)])

    @pl.when(sid == NS - 1)
    def _init_tail():
        pltpu.sync_copy(zer_hbm.at[pl.ds(0, TAILN)],
                        deg_sh.at[pl.ds(TAILO, TAILN)])

    pltpu.sync_copy(dst_hbm.at[wid], idx_v)
    pltpu.sync_copy(ones_hbm, ones_v)
    plsc.subcore_barrier()

    def body(j, carry):
        pltpu.sync_copy(ones_v, deg_sh.at[idx_v.at[j]], add=True)
        return carry

    lax.fori_loop(0, NCH, body, 0)
    plsc.subcore_barrier()
    pltpu.sync_copy(deg_sh.at[pl.ds(sid * SLAB, SLAB)],
                    out_hbm.at[cid, pl.ds(sid * SLAB, SLAB)])

    @pl.when(sid == NS - 1)
    def _out_tail():
        pltpu.sync_copy(deg_sh.at[pl.ds(TAILO, TAILN)],
                        out_hbm.at[cid, pl.ds(TAILO, TAILN)])


@functools.partial(
    pl.kernel,
    out_type=jax.ShapeDtypeStruct((NC, NN, DH), jnp.float32),
    mesh=_MESH,
    scratch_types=[
        pltpu.VMEM((2, SEGC, SCH), jnp.int32),
        pltpu.VMEM((2, SEGC, SCH), jnp.int32),
        pltpu.VMEM((SCH, DH), jnp.float32),
        pltpu.VMEM((SCH, DH), jnp.float32),
        pltpu.SemaphoreType.DMA,
        pltpu.SemaphoreType.DMA,
        pltpu.SemaphoreType.DMA,
        pltpu.SemaphoreType.DMA,
        pltpu.VMEM_SHARED((NN, DH), jnp.float32),
    ],
)
def _scatter_kernel(g_hbm, idx_hbm, zer_hbm, out_hbm,
                    sega, segb, rows0, rows1, sem0, sem1, semia, semib, s_sh):
    cid = lax.axis_index("c")
    sid = lax.axis_index("s")
    wid = cid * NS + sid
    pltpu.sync_copy(zer_hbm.at[pl.ds(0, SLAB)],
                    s_sh.at[pl.ds(sid * SLAB, SLAB)])

    @pl.when(sid == NS - 1)
    def _init_tail():
        pltpu.sync_copy(zer_hbm.at[pl.ds(0, TAILN)],
                        s_sh.at[pl.ds(TAILO, TAILN)])

    plsc.subcore_barrier()

    def run_seg(seg):
        pltpu.async_copy(g_hbm.at[seg.at[0, 0]], rows0, sem0)
        for j in range(SEGC):
            rows, sem = (rows0, sem0) if j % 2 == 0 else (rows1, sem1)
            if j + 1 < SEGC:
                nrows, nsem = (rows1, sem1) if j % 2 == 0 else (rows0, sem0)
                pltpu.async_copy(g_hbm.at[seg.at[0, j + 1]], nrows, nsem)
            pltpu.make_async_copy(g_hbm.at[seg.at[0, j]], rows, sem).wait()
            pltpu.sync_copy(rows, s_sh.at[seg.at[1, j]], add=True)

    pltpu.sync_copy(idx_hbm.at[wid, 0], sega)

    def seg_pair(t, carry):
        c0 = 2 * t
        pltpu.async_copy(idx_hbm.at[wid, c0 + 1], segb, semib)
        run_seg(sega)

        @pl.when(c0 + 2 < NSEG)
        def _pf():
            pltpu.async_copy(idx_hbm.at[wid, c0 + 2], sega, semia)

        pltpu.make_async_copy(idx_hbm.at[wid, c0 + 1], segb, semib).wait()
        run_seg(segb)

        @pl.when(c0 + 2 < NSEG)
        def _wf():
            pltpu.make_async_copy(idx_hbm.at[wid, c0 + 2], sega, semia).wait()

        return carry

    lax.fori_loop(0, NSEG // 2, seg_pair, 0)
    plsc.subcore_barrier()
    pltpu.sync_copy(s_sh.at[pl.ds(sid * SLAB, SLAB)],
                    out_hbm.at[cid, pl.ds(sid * SLAB, SLAB)])

    @pl.when(sid == NS - 1)
    def _out_tail():
        pltpu.sync_copy(s_sh.at[pl.ds(TAILO, TAILN)],
                        out_hbm.at[cid, pl.ds(TAILO, TAILN)])


_RB = 2000
_NB = NN // _RB


def _mm_scale_body(dega_ref, degb_ref, x_ref, w_ref, g_ref, dinv_ref):
    deg = dega_ref[:, 0:1] + degb_ref[:, 0:1] + 1.0
    di = lax.rsqrt(deg)
    g_ref[...] = jnp.dot(x_ref[...], w_ref[...],
                         preferred_element_type=jnp.float32) * di
    dinv_ref[...] = di


def _mm_scale(dega, degb, x, w):
    return pl.pallas_call(
        _mm_scale_body,
        grid=(_NB,),
        in_specs=[
            pl.BlockSpec((_RB, DEGW), lambda i: (i, 0)),
            pl.BlockSpec((_RB, DEGW), lambda i: (i, 0)),
            pl.BlockSpec((_RB, DH), lambda i: (i, 0)),
            pl.BlockSpec((DH, DH), lambda i: (0, 0)),
        ],
        out_specs=[
            pl.BlockSpec((_RB, DH), lambda i: (i, 0)),
            pl.BlockSpec((_RB, 1), lambda i: (i, 0)),
        ],
        out_shape=[
            jax.ShapeDtypeStruct((NN, DH), jnp.float32),
            jax.ShapeDtypeStruct((NN, 1), jnp.float32),
        ],
    )(dega, degb, x, w)


def _mid_body(s0_ref, s1_ref, g1_ref, dinv_ref, b_ref, w_ref, g2_ref):
    di = dinv_ref[...]
    h1 = jnp.maximum(
        di * (s0_ref[...] + s1_ref[...] + g1_ref[...]) + b_ref[...], 0.0)
    g2_ref[...] = jnp.dot(h1, w_ref[...],
                          preferred_element_type=jnp.float32) * di


def _mid(s0, s1, g1, dinv, b, w):
    return pl.pallas_call(
        _mid_body,
        grid=(_NB,),
        in_specs=[
            pl.BlockSpec((_RB, DH), lambda i: (i, 0)),
            pl.BlockSpec((_RB, DH), lambda i: (i, 0)),
            pl.BlockSpec((_RB, DH), lambda i: (i, 0)),
            pl.BlockSpec((_RB, 1), lambda i: (i, 0)),
            pl.BlockSpec((1, DH), lambda i: (0, 0)),
            pl.BlockSpec((DH, DH), lambda i: (0, 0)),
        ],
        out_specs=pl.BlockSpec((_RB, DH), lambda i: (i, 0)),
        out_shape=jax.ShapeDtypeStruct((NN, DH), jnp.float32),
    )(s0, s1, g1, dinv, b, w)


def _final_body(s0_ref, s1_ref, g2_ref, dinv_ref, b_ref, batch_ref,
                wout_ref, bout_ref, out_ref, acc, cnt):
    i = pl.program_id(0)

    @pl.when(i == 0)
    def _init():
        acc[...] = jnp.zeros_like(acc)
        cnt[...] = jnp.zeros_like(cnt)

    h2 = jnp.maximum(
        dinv_ref[...] * (s0_ref[...] + s1_ref[...] + g2_ref[...])
        + b_ref[...], 0.0)
    bt = batch_ref[0]
    m = (bt == lax.broadcasted_iota(jnp.int32, (NG, _RB), 0)
         ).astype(jnp.float32)
    acc[...] += jnp.dot(m, h2, preferred_element_type=jnp.float32)
    cnt[...] += jnp.sum(m, axis=1, keepdims=True)

    @pl.when(i == _NB - 1)
    def _fin():
        pooled = acc[...] / jnp.maximum(cnt[...], 1.0)
        out_ref[...] = jnp.dot(pooled, wout_ref[...],
                               preferred_element_type=jnp.float32) + bout_ref[...]


def _final(s0, s1, g2, dinv, b, batch3, wout, bout):
    return pl.pallas_call(
        _final_body,
        grid=(_NB,),
        in_specs=[
            pl.BlockSpec((_RB, DH), lambda i: (i, 0)),
            pl.BlockSpec((_RB, DH), lambda i: (i, 0)),
            pl.BlockSpec((_RB, DH), lambda i: (i, 0)),
            pl.BlockSpec((_RB, 1), lambda i: (i, 0)),
            pl.BlockSpec((1, DH), lambda i: (0, 0)),
            pl.BlockSpec((1, 1, _RB), lambda i: (i, 0, 0)),
            pl.BlockSpec((DH, DO), lambda i: (0, 0)),
            pl.BlockSpec((1, DO), lambda i: (0, 0)),
        ],
        out_specs=pl.BlockSpec((NG, DO), lambda i: (0, 0)),
        out_shape=jax.ShapeDtypeStruct((NG, DO), jnp.float32),
        scratch_shapes=[
            pltpu.VMEM((NG, DH), jnp.float32),
            pltpu.VMEM((NG, 1), jnp.float32),
        ],
    )(s0, s1, g2, dinv, b, batch3, wout, bout)


def kernel(x, edge_index, batch, W_in, b_in, W_h0, b_h0, W_out, b_out):
    idx5 = jnp.transpose(edge_index.reshape(2, NW, NSEG, SEGC, SCH),
                         (1, 2, 0, 3, 4))
    dst3 = edge_index[1].reshape(NW, NCH, CH)
    ones_deg = jnp.ones((CH, DEGW), jnp.float32)
    zer_deg = jnp.zeros((SLAB, DEGW), jnp.float32)
    zer_s = jnp.zeros((SLAB, DH), jnp.float32)

    deg2 = _deg_kernel(dst3, ones_deg, zer_deg)
    g1, dinv = _mm_scale(deg2[0], deg2[1], x, W_in)
    s1 = _scatter_kernel(g1, idx5, zer_s)
    g2 = _mid(s1[0], s1[1], g1, dinv, b_in.reshape(1, DH), W_h0)
    s2 = _scatter_kernel(g2, idx5, zer_s)
    return _final(s2[0], s2[1], g2, dinv, b_h0.reshape(1, DH),
                  batch.reshape(_NB, 1, _RB), W_out, b_out.reshape(1, DO))


# speedup vs baseline: 25.3582x; 1.0426x over previous
import functools

import jax
import jax.numpy as jnp
from jax import lax
from jax.experimental import pallas as pl
from jax.experimental.pallas import tpu as pltpu
from jax.experimental.pallas import tpu_sc as plsc

NN = 10000
NE = 320000
DH = 128
DO = 64
NG = 64

NC = 2
NS = 16
NW = NC * NS
EPT = NE // NW
CH = 100
NCH = EPT // CH
SCH = 125
SEGC = 8
NSEG = EPT // (SEGC * SCH)
SLAB = 624
TAILO = SLAB * NS
TAILN = NN - TAILO
DEGW = 128

_MESH = plsc.VectorSubcoreMesh(
    core_axis_name="c", subcore_axis_name="s", num_cores=NC, num_subcores=NS)


@functools.partial(
    pl.kernel,
    out_type=jax.ShapeDtypeStruct((NC, NN, DEGW), jnp.float32),
    mesh=_MESH,
    scratch_types=[
        pltpu.VMEM((NCH, CH), jnp.int32),
        pltpu.VMEM((CH, DEGW), jnp.float32),
        pltpu.SemaphoreType.DMA,
        pltpu.VMEM_SHARED((NN, DEGW), jnp.float32),
    ],
)
def _deg_kernel(dst_hbm, ones_hbm, zer_hbm, out_hbm, idx_v, ones_v, sem,
                deg_sh):
    cid = lax.axis_index("c")
    sid = lax.axis_index("s")
    wid = cid * NS + sid
    pltpu.sync_copy(zer_hbm.at[pl.ds(0, SLAB)],
                    deg_sh.at[pl.ds(sid * SLAB, SLAB)])

    @pl.when(sid == NS - 1)
    def _init_tail():
        pltpu.sync_copy(zer_hbm.at[pl.ds(0, TAILN)],
                        deg_sh.at[pl.ds(TAILO, TAILN)])

    pltpu.sync_copy(dst_hbm.at[wid], idx_v)
    pltpu.sync_copy(ones_hbm, ones_v)
    plsc.subcore_barrier()

    def body(q, carry):
        for u in range(4):
            pltpu.async_copy(ones_v, deg_sh.at[idx_v.at[4 * q + u]], sem,
                             add=True)
        for u in range(4):
            pltpu.make_async_copy(ones_v, deg_sh.at[idx_v.at[4 * q + u]],
                                  sem).wait()
        return carry

    lax.fori_loop(0, NCH // 4, body, 0)
    plsc.subcore_barrier()
    pltpu.sync_copy(deg_sh.at[pl.ds(sid * SLAB, SLAB)],
                    out_hbm.at[cid, pl.ds(sid * SLAB, SLAB)])

    @pl.when(sid == NS - 1)
    def _out_tail():
        pltpu.sync_copy(deg_sh.at[pl.ds(TAILO, TAILN)],
                        out_hbm.at[cid, pl.ds(TAILO, TAILN)])


@functools.partial(
    pl.kernel,
    out_type=jax.ShapeDtypeStruct((NC, NN, DH), jnp.float32),
    mesh=_MESH,
    scratch_types=[
        pltpu.VMEM((2, SEGC, SCH), jnp.int32),
        pltpu.VMEM((2, SEGC, SCH), jnp.int32),
        pltpu.VMEM((SCH, DH), jnp.float32),
        pltpu.VMEM((SCH, DH), jnp.float32),
        pltpu.SemaphoreType.DMA,
        pltpu.SemaphoreType.DMA,
        pltpu.SemaphoreType.DMA,
        pltpu.SemaphoreType.DMA,
        pltpu.VMEM_SHARED((NN, DH), jnp.float32),
    ],
)
def _scatter_kernel(g_hbm, idx_hbm, zer_hbm, out_hbm,
                    sega, segb, rows0, rows1, sem0, sem1, semia, semib, s_sh):
    cid = lax.axis_index("c")
    sid = lax.axis_index("s")
    wid = cid * NS + sid
    pltpu.sync_copy(zer_hbm.at[pl.ds(0, SLAB)],
                    s_sh.at[pl.ds(sid * SLAB, SLAB)])

    @pl.when(sid == NS - 1)
    def _init_tail():
        pltpu.sync_copy(zer_hbm.at[pl.ds(0, TAILN)],
                        s_sh.at[pl.ds(TAILO, TAILN)])

    plsc.subcore_barrier()

    def run_seg(seg):
        pltpu.async_copy(g_hbm.at[seg.at[0, 0]], rows0, sem0)
        for j in range(SEGC):
            rows, sem = (rows0, sem0) if j % 2 == 0 else (rows1, sem1)
            if j + 1 < SEGC:
                nrows, nsem = (rows1, sem1) if j % 2 == 0 else (rows0, sem0)
                pltpu.async_copy(g_hbm.at[seg.at[0, j + 1]], nrows, nsem)
            pltpu.make_async_copy(g_hbm.at[seg.at[0, j]], rows, sem).wait()
            pltpu.sync_copy(rows, s_sh.at[seg.at[1, j]], add=True)

    pltpu.sync_copy(idx_hbm.at[wid, 0], sega)

    def seg_pair(t, carry):
        c0 = 2 * t
        pltpu.async_copy(idx_hbm.at[wid, c0 + 1], segb, semib)
        run_seg(sega)

        @pl.when(c0 + 2 < NSEG)
        def _pf():
            pltpu.async_copy(idx_hbm.at[wid, c0 + 2], sega, semia)

        pltpu.make_async_copy(idx_hbm.at[wid, c0 + 1], segb, semib).wait()
        run_seg(segb)

        @pl.when(c0 + 2 < NSEG)
        def _wf():
            pltpu.make_async_copy(idx_hbm.at[wid, c0 + 2], sega, semia).wait()

        return carry

    lax.fori_loop(0, NSEG // 2, seg_pair, 0)
    plsc.subcore_barrier()
    pltpu.sync_copy(s_sh.at[pl.ds(sid * SLAB, SLAB)],
                    out_hbm.at[cid, pl.ds(sid * SLAB, SLAB)])

    @pl.when(sid == NS - 1)
    def _out_tail():
        pltpu.sync_copy(s_sh.at[pl.ds(TAILO, TAILN)],
                        out_hbm.at[cid, pl.ds(TAILO, TAILN)])


_RB = 2000
_NB = NN // _RB


def _mm_scale_body(dega_ref, degb_ref, x_ref, w_ref, g_ref, dinv_ref):
    deg = dega_ref[:, 0:1] + degb_ref[:, 0:1] + 1.0
    di = lax.rsqrt(deg)
    g_ref[...] = jnp.dot(x_ref[...], w_ref[...],
                         preferred_element_type=jnp.float32) * di
    dinv_ref[...] = di


def _mm_scale(dega, degb, x, w):
    return pl.pallas_call(
        _mm_scale_body,
        grid=(_NB,),
        in_specs=[
            pl.BlockSpec((_RB, 8), lambda i: (i, 0)),
            pl.BlockSpec((_RB, 8), lambda i: (i, 0)),
            pl.BlockSpec((_RB, DH), lambda i: (i, 0)),
            pl.BlockSpec((DH, DH), lambda i: (0, 0)),
        ],
        out_specs=[
            pl.BlockSpec((_RB, DH), lambda i: (i, 0)),
            pl.BlockSpec((_RB, 1), lambda i: (i, 0)),
        ],
        out_shape=[
            jax.ShapeDtypeStruct((NN, DH), jnp.float32),
            jax.ShapeDtypeStruct((NN, 1), jnp.float32),
        ],
    )(dega, degb, x, w)


def _mid_body(s0_ref, s1_ref, g1_ref, dinv_ref, b_ref, w_ref, g2_ref):
    di = dinv_ref[...]
    h1 = jnp.maximum(
        di * (s0_ref[...] + s1_ref[...] + g1_ref[...]) + b_ref[...], 0.0)
    g2_ref[...] = jnp.dot(h1, w_ref[...],
                          preferred_element_type=jnp.float32) * di


def _mid(s0, s1, g1, dinv, b, w):
    return pl.pallas_call(
        _mid_body,
        grid=(_NB,),
        in_specs=[
            pl.BlockSpec((_RB, DH), lambda i: (i, 0)),
            pl.BlockSpec((_RB, DH), lambda i: (i, 0)),
            pl.BlockSpec((_RB, DH), lambda i: (i, 0)),
            pl.BlockSpec((_RB, 1), lambda i: (i, 0)),
            pl.BlockSpec((1, DH), lambda i: (0, 0)),
            pl.BlockSpec((DH, DH), lambda i: (0, 0)),
        ],
        out_specs=pl.BlockSpec((_RB, DH), lambda i: (i, 0)),
        out_shape=jax.ShapeDtypeStruct((NN, DH), jnp.float32),
    )(s0, s1, g1, dinv, b, w)


def _final_body(s0_ref, s1_ref, g2_ref, dinv_ref, b_ref, batch_ref,
                wout_ref, bout_ref, out_ref, acc, cnt):
    i = pl.program_id(0)

    @pl.when(i == 0)
    def _init():
        acc[...] = jnp.zeros_like(acc)
        cnt[...] = jnp.zeros_like(cnt)

    h2 = jnp.maximum(
        dinv_ref[...] * (s0_ref[...] + s1_ref[...] + g2_ref[...])
        + b_ref[...], 0.0)
    bt = batch_ref[0]
    m = (bt == lax.broadcasted_iota(jnp.int32, (NG, _RB), 0)
         ).astype(jnp.float32)
    acc[...] += jnp.dot(m, h2, preferred_element_type=jnp.float32)
    cnt[...] += jnp.sum(m, axis=1, keepdims=True)

    @pl.when(i == _NB - 1)
    def _fin():
        pooled = acc[...] / jnp.maximum(cnt[...], 1.0)
        out_ref[...] = jnp.dot(pooled, wout_ref[...],
                               preferred_element_type=jnp.float32) + bout_ref[...]


def _final(s0, s1, g2, dinv, b, batch3, wout, bout):
    return pl.pallas_call(
        _final_body,
        grid=(_NB,),
        in_specs=[
            pl.BlockSpec((_RB, DH), lambda i: (i, 0)),
            pl.BlockSpec((_RB, DH), lambda i: (i, 0)),
            pl.BlockSpec((_RB, DH), lambda i: (i, 0)),
            pl.BlockSpec((_RB, 1), lambda i: (i, 0)),
            pl.BlockSpec((1, DH), lambda i: (0, 0)),
            pl.BlockSpec((1, 1, _RB), lambda i: (i, 0, 0)),
            pl.BlockSpec((DH, DO), lambda i: (0, 0)),
            pl.BlockSpec((1, DO), lambda i: (0, 0)),
        ],
        out_specs=pl.BlockSpec((NG, DO), lambda i: (0, 0)),
        out_shape=jax.ShapeDtypeStruct((NG, DO), jnp.float32),
        scratch_shapes=[
            pltpu.VMEM((NG, DH), jnp.float32),
            pltpu.VMEM((NG, 1), jnp.float32),
        ],
    )(s0, s1, g2, dinv, b, batch3, wout, bout)


def kernel(x, edge_index, batch, W_in, b_in, W_h0, b_h0, W_out, b_out):
    idx5 = jnp.transpose(edge_index.reshape(2, NW, NSEG, SEGC, SCH),
                         (1, 2, 0, 3, 4))
    dst3 = edge_index[1].reshape(NW, NCH, CH)
    ones_deg = jnp.ones((CH, DEGW), jnp.float32)
    zer_deg = jnp.zeros((SLAB, DEGW), jnp.float32)
    zer_s = jnp.zeros((SLAB, DH), jnp.float32)

    deg2 = _deg_kernel(dst3, ones_deg, zer_deg)
    g1, dinv = _mm_scale(deg2[0, :, :8], deg2[1, :, :8], x, W_in)
    s1 = _scatter_kernel(g1, idx5, zer_s)
    g2 = _mid(s1[0], s1[1], g1, dinv, b_in.reshape(1, DH), W_h0)
    s2 = _scatter_kernel(g2, idx5, zer_s)
    return _final(s2[0], s2[1], g2, dinv, b_h0.reshape(1, DH),
                  batch.reshape(_NB, 1, _RB), W_out, b_out.reshape(1, DO))
